# Initial kernel scaffold; baseline (speedup 1.0000x reference)
import jax
import jax.numpy as jnp
from jax.experimental import pallas as pl


def kernel(x, edge_index, W_agg, b_agg, W_self, b_self):
    raise NotImplementedError("write your pallas kernel here")


# speedup vs baseline: 2.6726x; 2.6726x over previous
import jax
import jax.numpy as jnp
from jax import lax
from jax.experimental import pallas as pl
from jax.experimental.pallas import tpu as pltpu
from jax.experimental.pallas import tpu_sc as plsc

N_NODES = 10000
N_EDGES = 320000
D = 128
NC, NS = 2, 16
HALF = N_NODES // NC
TRASH = HALF
NACC = 5120
EPW = N_EDGES // NS
CH = 80
SCH = 10
NSUP = EPW // (SCH * CH)
RPS = NACC // NS
CGR = NACC // 16
RTILES = [(t * CH, CH) for t in range(RPS // CH)]
VPC = CH // 16


def _sc_agg(x_hbm, src_hbm, dst_hbm, z128_hbm,
            sum_out,
            sum_sh, src_v, dst_v, rows_v, gsem):
    c = lax.axis_index("c")
    s = lax.axis_index("s")
    base = pl.multiple_of(s * RPS, 8)
    lo = c * HALF

    pltpu.sync_copy(z128_hbm, rows_v)
    for off, sz in RTILES:
        pltpu.sync_copy(rows_v, sum_sh.at[pl.ds(base + off, sz)])

    plsc.subcore_barrier()

    def outer(k, carry):
        pltpu.sync_copy(src_hbm.at[s, k], src_v)
        pltpu.sync_copy(dst_hbm.at[s, k], dst_v)

        def remap(t, rcarry):
            i = t // VPC
            col = (t % VPC) * 16
            v = dst_v[i, pl.ds(col, 16)]
            d = v - lo
            keep = (d >= 0) & (d < HALF)
            d = jnp.where(keep, d, TRASH)
            dst_v[i, pl.ds(col, 16)] = d
            return rcarry

        lax.fori_loop(0, SCH * VPC, remap, carry)

        def body(j, inner_carry):
            pltpu.async_copy(x_hbm.at[src_v.at[j]], rows_v, gsem).wait()
            pltpu.sync_copy(rows_v, sum_sh.at[dst_v.at[j]], add=True)
            return inner_carry

        return lax.fori_loop(0, SCH, body, carry)

    lax.fori_loop(0, NSUP, outer, 0)
    plsc.subcore_barrier()

    for off, sz in RTILES:
        sl = pl.ds(base + off, sz)
        pltpu.sync_copy(sum_sh.at[sl], rows_v)
        pltpu.sync_copy(rows_v, sum_out.at[c, sl])


_agg_call_cache = None


def _agg_call(*args):
    global _agg_call_cache
    if _agg_call_cache is None:
        _agg_call_cache = _make_agg_call()
    return _agg_call_cache(*args)


def _make_agg_call():
    return pl.kernel(
        _sc_agg,
        out_type=jax.ShapeDtypeStruct((NC, NACC, D), jnp.float32),
        mesh=plsc.VectorSubcoreMesh(core_axis_name="c", subcore_axis_name="s",
                                    num_cores=NC, num_subcores=NS),
        scratch_types=[
            pltpu.VMEM_SHARED((NACC, D), jnp.float32),
            pltpu.VMEM((SCH, CH), jnp.int32),
            pltpu.VMEM((SCH, CH), jnp.int32),
            pltpu.VMEM((CH, D), jnp.float32),
            pltpu.SemaphoreType.DMA,
        ],
    )


ROWS_TC = 640
NPADTC = NC * NACC


def _tc_body(psum_ref, pcnt_ref, x_ref, wa_ref, ba_ref, ws_ref, bs_ref,
             out_ref):
    cnt = pcnt_ref[...]
    mean = psum_ref[...] / jnp.maximum(cnt, 1.0)
    h_agg = jnp.dot(mean, wa_ref[...],
                    preferred_element_type=jnp.float32) + ba_ref[...]
    h_self = jnp.dot(x_ref[...], ws_ref[...],
                     preferred_element_type=jnp.float32) + bs_ref[...]
    out_ref[:, :D] = jnp.maximum(h_agg, 0.0)
    out_ref[:, D:] = jnp.maximum(h_self, 0.0)


_tc_call = pl.pallas_call(
    _tc_body,
    grid=(NPADTC // ROWS_TC,),
    in_specs=[
        pl.BlockSpec((ROWS_TC, D), lambda i: (i, 0)),
        pl.BlockSpec((ROWS_TC, 1), lambda i: (i, 0)),
        pl.BlockSpec((ROWS_TC, D), lambda i: (i, 0)),
        pl.BlockSpec((D, D), lambda i: (0, 0)),
        pl.BlockSpec((1, D), lambda i: (0, 0)),
        pl.BlockSpec((D, D), lambda i: (0, 0)),
        pl.BlockSpec((1, D), lambda i: (0, 0)),
    ],
    out_specs=pl.BlockSpec((ROWS_TC, 2 * D), lambda i: (i, 0)),
    out_shape=jax.ShapeDtypeStruct((NPADTC, 2 * D), jnp.float32),
)


def kernel(x, edge_index, W_agg, b_agg, W_self, b_self):
    src = edge_index[0].reshape(NS, NSUP, SCH, CH)
    dst = edge_index[1].reshape(NS, NSUP, SCH, CH)
    z128 = jnp.zeros((CH, D), jnp.float32)
    psum = _agg_call(x, src, dst, z128)
    counts = jax.ops.segment_sum(
        jnp.ones((N_EDGES,), jnp.float32), edge_index[1],
        num_segments=N_NODES)
    zc = jnp.zeros((NACC - HALF,), jnp.float32)
    cnt = jnp.concatenate(
        [counts[:HALF], zc, counts[HALF:], zc]).reshape(NPADTC, 1)
    psum_flat = psum.reshape(NPADTC, D)
    zpad = jnp.zeros((NACC - HALF, D), jnp.float32)
    x_pad = jnp.concatenate([x[:HALF], zpad, x[HALF:], zpad], axis=0)
    out_pad = _tc_call(psum_flat, cnt, x_pad, W_agg, b_agg.reshape(1, D),
                       W_self, b_self.reshape(1, D))
    return jnp.concatenate(
        [out_pad[:HALF], out_pad[NACC:NACC + HALF]], axis=0)
